# baseline (speedup 1.0000x reference)
import functools

import jax
import jax.numpy as jnp
from jax import lax
from jax.experimental import pallas as pl
from jax.experimental.pallas import tpu as pltpu
from jax.experimental.pallas import tpu_sc as plsc

N = 10000
R = 64
D = 128
E = 320000

NC = 2
NS = 16
NW = NC * NS
CB = 128
CH = 90
G = 10
K0 = 120
K1 = 40
TCH = NS * (K0 + K1)
EP = TCH * CB
NA = 10112
RPT = NA // NS
CW = 16
BN = 1000


def _idx_body(et_ref, src_ref, o_ref):
    et = et_ref[...]
    cls = (et >= R).astype(jnp.int32) + (et >= 2 * R).astype(jnp.int32)
    o_ref[...] = cls * N + src_ref[...]


def _make_gidx(etp, srcp):
    rows = EP // 128
    return pl.pallas_call(
        _idx_body,
        out_shape=jax.ShapeDtypeStruct((rows, 128), jnp.int32),
    )(etp.reshape(rows, 128), srcp.reshape(rows, 128))


def _proj_body(x_ref, wt_ref, b_ref, o_ref):
    y = jnp.dot(x_ref[...], wt_ref[0], preferred_element_type=jnp.float32)
    o_ref[...] = (y + b_ref[0])[None]


def _proj(x, wt3, b3):
    return pl.pallas_call(
        _proj_body,
        grid=(3, N // BN),
        in_specs=[
            pl.BlockSpec((BN, D), lambda c, i: (i, 0)),
            pl.BlockSpec((1, D, D), lambda c, i: (c, 0, 0)),
            pl.BlockSpec((1, 1, D), lambda c, i: (c, 0, 0)),
        ],
        out_specs=pl.BlockSpec((1, BN, D), lambda c, i: (c, i, 0)),
        out_shape=jax.ShapeDtypeStruct((3, N, D), jnp.float32),
    )(x, wt3, b3.reshape(3, 1, D))


def _sc_body(gidx_hbm, dst_hbm, y_hbm, z128_hbm, z16_hbm, ones_hbm,
             a_out, c_out,
             gidx_g, dst_g, rows_v, ones_v, a_sh, c_sh,
             gsem, isem, ssem):
    cid = lax.axis_index("c")
    sid = lax.axis_index("s")
    k_my = lax.select(cid == 0, K0, K1)
    ng_my = k_my // G
    cstart = lax.select(cid == 0, sid * K0, NS * K0 + sid * K1)

    pltpu.sync_copy(ones_hbm, ones_v)
    row0 = sid * RPT
    pltpu.sync_copy(z128_hbm, a_sh.at[pl.ds(row0, RPT)])
    pltpu.sync_copy(z16_hbm, c_sh.at[pl.ds(row0, RPT)])
    plsc.subcore_barrier()

    def pre(g, gb):
        pltpu.async_copy(
            gidx_hbm.at[pl.ds(cstart + g * G, G)], gidx_g.at[gb], isem)
        pltpu.async_copy(
            dst_hbm.at[pl.ds(cstart + g * G, G)], dst_g.at[gb], isem)

    def iwait():
        pltpu.make_async_copy(
            gidx_hbm.at[pl.ds(0, G)], gidx_g.at[0], isem).wait()
        pltpu.make_async_copy(
            dst_hbm.at[pl.ds(0, G)], dst_g.at[0], isem).wait()

    def start(j, b):
        gb = (j // G) % 2
        pltpu.async_copy(y_hbm.at[gidx_g.at[gb, j % G]], rows_v.at[b], gsem)

    def wait(b):
        pltpu.make_async_copy(
            y_hbm.at[gidx_g.at[0, 0]], rows_v.at[b], gsem).wait()

    def ascat(j, b):
        gb = (j // G) % 2
        didx = dst_g.at[gb, j % G]
        pltpu.async_copy(rows_v.at[b], a_sh.at[didx], ssem, add=True)
        pltpu.async_copy(ones_v, c_sh.at[didx], ssem, add=True)

    def sdrain():
        pltpu.make_async_copy(rows_v.at[0], a_sh.at[dst_g.at[0, 0]], ssem).wait()
        pltpu.make_async_copy(ones_v, c_sh.at[dst_g.at[0, 0]], ssem).wait()

    pre(0, 0)
    iwait()
    start(0, 0)

    def mloop(k, carry):
        g = k // (G // 2)
        r = k % (G // 2)
        j0 = 2 * k

        @pl.when(jnp.logical_and(r == 0, g + 1 < ng_my))
        def _():
            pre(g + 1, (g + 1) % 2)

        wait(0)
        start(j0 + 1, 1)
        ascat(j0, 0)

        @pl.when(jnp.logical_and(r == G // 2 - 1, g + 1 < ng_my))
        def _():
            iwait()

        wait(1)
        sdrain()

        @pl.when(j0 + 2 < k_my)
        def _():
            start(j0 + 2, 0)

        ascat(j0 + 1, 1)
        sdrain()
        return carry

    lax.fori_loop(0, k_my // 2, mloop, 0)

    plsc.subcore_barrier()
    pltpu.sync_copy(a_sh.at[pl.ds(row0, RPT)], a_out.at[cid, pl.ds(row0, RPT)])
    pltpu.sync_copy(c_sh.at[pl.ds(row0, RPT)], c_out.at[cid, pl.ds(row0, RPT)])


def _sc_aggregate(gidx3, dst3, y, z128, z16, ones16):
    mesh = plsc.VectorSubcoreMesh(core_axis_name="c", subcore_axis_name="s")
    f = functools.partial(
        pl.kernel,
        out_type=(
            jax.ShapeDtypeStruct((NC, NA, D), jnp.float32),
            jax.ShapeDtypeStruct((NC, NA, CW), jnp.float32),
        ),
        mesh=mesh,
        scratch_types=[
            pltpu.VMEM((2, G, CB), jnp.int32),
            pltpu.VMEM((2, G, CB), jnp.int32),
            pltpu.VMEM((2, CB, D), jnp.float32),
            pltpu.VMEM((CB, CW), jnp.float32),
            pltpu.VMEM_SHARED((NA, D), jnp.float32),
            pltpu.VMEM_SHARED((NA, CW), jnp.float32),
            pltpu.SemaphoreType.DMA,
            pltpu.SemaphoreType.DMA,
            pltpu.SemaphoreType.DMA,
        ],
        compiler_params=pltpu.CompilerParams(use_tc_tiling_on_sc=False),
    )(_sc_body)
    return f(gidx3, dst3, y, z128, z16, ones16)


def _post_body(p_ref, pc_ref, x_ref, wsk_ref, bsk_ref, g_ref, be_ref, o_ref):
    s = p_ref[0] + p_ref[1]
    pc = pc_ref[0] + pc_ref[1]
    cnt = jnp.sum(pc, axis=1, keepdims=True) * (1.0 / CW)
    cnt = jnp.maximum(cnt, 1.0)
    h = jnp.maximum(s / cnt, 0.0)
    y = h + jnp.dot(x_ref[...], wsk_ref[...], preferred_element_type=jnp.float32)
    y = y + bsk_ref[...]
    mu = jnp.mean(y, axis=1, keepdims=True)
    d = y - mu
    var = jnp.mean(d * d, axis=1, keepdims=True)
    o_ref[...] = g_ref[...] * (d * lax.rsqrt(var + 1e-5)) + be_ref[...]


def _post(p, pc, x, wskt, bsk, g, be):
    return pl.pallas_call(
        _post_body,
        grid=(N // BN,),
        in_specs=[
            pl.BlockSpec((NC, BN, D), lambda i: (0, i, 0)),
            pl.BlockSpec((NC, BN, CW), lambda i: (0, i, 0)),
            pl.BlockSpec((BN, D), lambda i: (i, 0)),
            pl.BlockSpec((D, D), lambda i: (0, 0)),
            pl.BlockSpec((1, D), lambda i: (0, 0)),
            pl.BlockSpec((1, D), lambda i: (0, 0)),
            pl.BlockSpec((1, D), lambda i: (0, 0)),
        ],
        out_specs=pl.BlockSpec((BN, D), lambda i: (i, 0)),
        out_shape=jax.ShapeDtypeStruct((N, D), jnp.float32),
    )(p, pc, x, wskt, bsk.reshape(1, D), g.reshape(1, D), be.reshape(1, D))


def _layer(x, gidx3, dst3, consts, wt3, b3, wskt, bsk, g, be):
    z128, z16, ones16 = consts
    y = _proj(x, wt3, b3).reshape(3 * N, D)
    p, pc = _sc_aggregate(gidx3, dst3, y, z128, z16, ones16)
    return _post(p, pc, x, wskt, bsk, g, be)


def kernel(edges, node_emb, W1o, b1o, W1i, b1i, W1s, b1s, g1, be1, Wsk1, bsk1,
           W2o, b2o, W2i, b2i, W2s, b2s, g2, be2, Wsk2, bsk2):
    src = edges[:, 0]
    et = edges[:, 1]
    dst = edges[:, 2]
    pad = EP - E
    srcp = jnp.concatenate([src, jnp.zeros((pad,), jnp.int32)])
    etp = jnp.concatenate([et, jnp.zeros((pad,), jnp.int32)])
    dstp = jnp.concatenate([dst, jnp.full((pad,), N, jnp.int32)])

    gidx3 = _make_gidx(etp, srcp).reshape(TCH, CB)
    dst3 = dstp.reshape(TCH, CB)

    consts = (
        jnp.zeros((RPT, D), jnp.float32),
        jnp.zeros((RPT, CW), jnp.float32),
        jnp.ones((CB, CW), jnp.float32),
    )

    wt1 = jnp.stack([W1o.T, W1i.T, W1s.T])
    b1 = jnp.stack([b1o, b1i, b1s])
    wt2 = jnp.stack([W2o.T, W2i.T, W2s.T])
    b2 = jnp.stack([b2o, b2i, b2s])

    x1 = _layer(node_emb, gidx3, dst3, consts, wt1, b1, Wsk1.T, bsk1, g1, be1)
    x2 = _layer(x1, gidx3, dst3, consts, wt2, b2, Wsk2.T, bsk2, g2, be2)
    return x2


# speedup vs baseline: 1.7297x; 1.7297x over previous
import functools

import jax
import jax.numpy as jnp
from jax import lax
from jax.experimental import pallas as pl
from jax.experimental.pallas import tpu as pltpu
from jax.experimental.pallas import tpu_sc as plsc

N = 10000
R = 64
D = 128
E = 320000

NC = 2
NS = 16
NW = NC * NS
CB = 112
CH = 90
G = 10
K0 = 130
K1 = 50
TCH = NS * (K0 + K1)
EP = TCH * CB
NA = 10112
RPT = NA // NS
CW = 16
BN = 1000


def _idx_body(et_ref, src_ref, o_ref):
    et = et_ref[...]
    cls = (et >= R).astype(jnp.int32) + (et >= 2 * R).astype(jnp.int32)
    o_ref[...] = cls * N + src_ref[...]


def _make_gidx(etp, srcp):
    rows = EP // 128
    return pl.pallas_call(
        _idx_body,
        out_shape=jax.ShapeDtypeStruct((rows, 128), jnp.int32),
    )(etp.reshape(rows, 128), srcp.reshape(rows, 128))


def _proj_body(x_ref, wt_ref, b_ref, o_ref):
    y = jnp.dot(x_ref[...], wt_ref[0], preferred_element_type=jnp.float32)
    o_ref[...] = (y + b_ref[0])[None]


def _proj(x, wt3, b3):
    return pl.pallas_call(
        _proj_body,
        grid=(3, N // BN),
        in_specs=[
            pl.BlockSpec((BN, D), lambda c, i: (i, 0)),
            pl.BlockSpec((1, D, D), lambda c, i: (c, 0, 0)),
            pl.BlockSpec((1, 1, D), lambda c, i: (c, 0, 0)),
        ],
        out_specs=pl.BlockSpec((1, BN, D), lambda c, i: (c, i, 0)),
        out_shape=jax.ShapeDtypeStruct((3, N, D), jnp.float32),
    )(x, wt3, b3.reshape(3, 1, D))


def _sc_body(gidx_hbm, dst_hbm, y_hbm, z128_hbm, z16_hbm, ones_hbm,
             a_out, c_out,
             gidx_g, dst_g, rows_v, ones_v, a_sh, c_sh,
             gsem, isem, ssem):
    cid = lax.axis_index("c")
    sid = lax.axis_index("s")
    k_my = lax.select(cid == 0, K0, K1)
    ng_my = k_my // G
    cstart = lax.select(cid == 0, sid * K0, NS * K0 + sid * K1)

    pltpu.sync_copy(ones_hbm, ones_v)
    row0 = sid * RPT
    pltpu.sync_copy(z128_hbm, a_sh.at[pl.ds(row0, RPT)])
    pltpu.sync_copy(z16_hbm, c_sh.at[pl.ds(row0, RPT)])
    plsc.subcore_barrier()

    def pre(g, gb):
        pltpu.async_copy(
            gidx_hbm.at[pl.ds(cstart + g * G, G)], gidx_g.at[gb], isem)
        pltpu.async_copy(
            dst_hbm.at[pl.ds(cstart + g * G, G)], dst_g.at[gb], isem)

    def iwait():
        pltpu.make_async_copy(
            gidx_hbm.at[pl.ds(0, G)], gidx_g.at[0], isem).wait()
        pltpu.make_async_copy(
            dst_hbm.at[pl.ds(0, G)], dst_g.at[0], isem).wait()

    def start(j, b):
        gb = (j // G) % 2
        pltpu.async_copy(y_hbm.at[gidx_g.at[gb, j % G]], rows_v.at[b], gsem)

    def wait(b):
        pltpu.make_async_copy(
            y_hbm.at[gidx_g.at[0, 0]], rows_v.at[b], gsem).wait()

    def ascat(j, b):
        gb = (j // G) % 2
        didx = dst_g.at[gb, j % G]
        pltpu.async_copy(rows_v.at[b], a_sh.at[didx], ssem, add=True)
        pltpu.async_copy(ones_v, c_sh.at[didx], ssem, add=True)

    def sdrain():
        pltpu.make_async_copy(rows_v.at[0], a_sh.at[dst_g.at[0, 0]], ssem).wait()
        pltpu.make_async_copy(ones_v, c_sh.at[dst_g.at[0, 0]], ssem).wait()

    pre(0, 0)
    iwait()
    start(0, 0)

    def mloop(k, carry):
        g = k // (G // 2)
        r = k % (G // 2)
        j0 = 2 * k

        @pl.when(jnp.logical_and(r == 0, g + 1 < ng_my))
        def _():
            pre(g + 1, (g + 1) % 2)

        wait(0)
        start(j0 + 1, 1)
        ascat(j0, 0)

        @pl.when(jnp.logical_and(r == G // 2 - 1, g + 1 < ng_my))
        def _():
            iwait()

        wait(1)
        sdrain()

        @pl.when(j0 + 2 < k_my)
        def _():
            start(j0 + 2, 0)

        ascat(j0 + 1, 1)
        sdrain()
        return carry

    lax.fori_loop(0, k_my // 2, mloop, 0)

    plsc.subcore_barrier()
    pltpu.sync_copy(a_sh.at[pl.ds(row0, RPT)], a_out.at[cid, pl.ds(row0, RPT)])
    pltpu.sync_copy(c_sh.at[pl.ds(row0, RPT)], c_out.at[cid, pl.ds(row0, RPT)])


def _sc_aggregate(gidx3, dst3, y, z128, z16, ones16):
    mesh = plsc.VectorSubcoreMesh(core_axis_name="c", subcore_axis_name="s")
    f = functools.partial(
        pl.kernel,
        out_type=(
            jax.ShapeDtypeStruct((NC, NA, D), jnp.float32),
            jax.ShapeDtypeStruct((NC, NA, CW), jnp.float32),
        ),
        mesh=mesh,
        scratch_types=[
            pltpu.VMEM((2, G, CB), jnp.int32),
            pltpu.VMEM((2, G, CB), jnp.int32),
            pltpu.VMEM((2, CB, D), jnp.float32),
            pltpu.VMEM((CB, CW), jnp.float32),
            pltpu.VMEM_SHARED((NA, D), jnp.float32),
            pltpu.VMEM_SHARED((NA, CW), jnp.float32),
            pltpu.SemaphoreType.DMA,
            pltpu.SemaphoreType.DMA,
            pltpu.SemaphoreType.DMA,
        ],
        compiler_params=pltpu.CompilerParams(use_tc_tiling_on_sc=False),
    )(_sc_body)
    return f(gidx3, dst3, y, z128, z16, ones16)


def _post_body(p_ref, pc_ref, x_ref, wsk_ref, bsk_ref, g_ref, be_ref, o_ref):
    s = p_ref[0] + p_ref[1]
    pc = pc_ref[0] + pc_ref[1]
    cnt = jnp.sum(pc, axis=1, keepdims=True) * (1.0 / CW)
    cnt = jnp.maximum(cnt, 1.0)
    h = jnp.maximum(s / cnt, 0.0)
    y = h + jnp.dot(x_ref[...], wsk_ref[...], preferred_element_type=jnp.float32)
    y = y + bsk_ref[...]
    mu = jnp.mean(y, axis=1, keepdims=True)
    d = y - mu
    var = jnp.mean(d * d, axis=1, keepdims=True)
    o_ref[...] = g_ref[...] * (d * lax.rsqrt(var + 1e-5)) + be_ref[...]


def _post(p, pc, x, wskt, bsk, g, be):
    return pl.pallas_call(
        _post_body,
        grid=(N // BN,),
        in_specs=[
            pl.BlockSpec((NC, BN, D), lambda i: (0, i, 0)),
            pl.BlockSpec((NC, BN, CW), lambda i: (0, i, 0)),
            pl.BlockSpec((BN, D), lambda i: (i, 0)),
            pl.BlockSpec((D, D), lambda i: (0, 0)),
            pl.BlockSpec((1, D), lambda i: (0, 0)),
            pl.BlockSpec((1, D), lambda i: (0, 0)),
            pl.BlockSpec((1, D), lambda i: (0, 0)),
        ],
        out_specs=pl.BlockSpec((BN, D), lambda i: (i, 0)),
        out_shape=jax.ShapeDtypeStruct((N, D), jnp.float32),
    )(p, pc, x, wskt, bsk.reshape(1, D), g.reshape(1, D), be.reshape(1, D))


def _layer(x, gidx3, dst3, consts, wt3, b3, wskt, bsk, g, be):
    z128, z16, ones16 = consts
    y = _proj(x, wt3, b3).reshape(3 * N, D)
    p, pc = _sc_aggregate(gidx3, dst3, y, z128, z16, ones16)
    return _post(p, pc, x, wskt, bsk, g, be)


def kernel(edges, node_emb, W1o, b1o, W1i, b1i, W1s, b1s, g1, be1, Wsk1, bsk1,
           W2o, b2o, W2i, b2i, W2s, b2s, g2, be2, Wsk2, bsk2):
    src = edges[:, 0]
    et = edges[:, 1]
    dst = edges[:, 2]
    pad = EP - E
    srcp = jnp.concatenate([src, jnp.zeros((pad,), jnp.int32)])
    etp = jnp.concatenate([et, jnp.zeros((pad,), jnp.int32)])
    dstp = jnp.concatenate([dst, jnp.full((pad,), N, jnp.int32)])

    gidx3 = _make_gidx(etp, srcp).reshape(TCH, CB)
    dst3 = dstp.reshape(TCH, CB)

    consts = (
        jnp.zeros((RPT, D), jnp.float32),
        jnp.zeros((RPT, CW), jnp.float32),
        jnp.ones((CB, CW), jnp.float32),
    )

    wt1 = jnp.stack([W1o.T, W1i.T, W1s.T])
    b1 = jnp.stack([b1o, b1i, b1s])
    wt2 = jnp.stack([W2o.T, W2i.T, W2s.T])
    b2 = jnp.stack([b2o, b2i, b2s])

    x1 = _layer(node_emb, gidx3, dst3, consts, wt1, b1, Wsk1.T, bsk1, g1, be1)
    x2 = _layer(x1, gidx3, dst3, consts, wt2, b2, Wsk2.T, bsk2, g2, be2)
    return x2


# speedup vs baseline: 1.7901x; 1.0349x over previous
import functools

import jax
import jax.numpy as jnp
from jax import lax
from jax.experimental import pallas as pl
from jax.experimental.pallas import tpu as pltpu
from jax.experimental.pallas import tpu_sc as plsc

N = 10000
R = 64
D = 128
E = 320000

NC = 2
NS = 16
NW = NC * NS
CB = 112
CH = 90
G = 10
K0 = 130
K1 = 50
TCH = NS * (K0 + K1)
EP = TCH * CB
NA = 10112
RPT = NA // NS
CW = 16
BN = 1000


def _idx_body(et_ref, src_ref, o_ref):
    et = et_ref[...]
    cls = (et >= R).astype(jnp.int32) + (et >= 2 * R).astype(jnp.int32)
    o_ref[...] = cls * N + src_ref[...]


def _make_gidx(etp, srcp):
    rows = EP // 128
    return pl.pallas_call(
        _idx_body,
        out_shape=jax.ShapeDtypeStruct((rows, 128), jnp.int32),
    )(etp.reshape(rows, 128), srcp.reshape(rows, 128))


def _proj_body(x_ref, wt_ref, b_ref, o_ref):
    y = jnp.dot(x_ref[...], wt_ref[0], preferred_element_type=jnp.float32)
    o_ref[...] = (y + b_ref[0])[None]


def _proj(x, wt3, b3):
    return pl.pallas_call(
        _proj_body,
        grid=(3, N // BN),
        in_specs=[
            pl.BlockSpec((BN, D), lambda c, i: (i, 0)),
            pl.BlockSpec((1, D, D), lambda c, i: (c, 0, 0)),
            pl.BlockSpec((1, 1, D), lambda c, i: (c, 0, 0)),
        ],
        out_specs=pl.BlockSpec((1, BN, D), lambda c, i: (c, i, 0)),
        out_shape=jax.ShapeDtypeStruct((3, N, D), jnp.float32),
    )(x, wt3, b3.reshape(3, 1, D))


def _sc_impl(with_counts, gidx_hbm, dst_hbm, y_hbm, z128_hbm, z16_hbm,
             ones_hbm, a_out, c_out,
             gidx_g, dst_g, rows_v, ones_v, a_sh, c_sh,
             gsem, isem, ssem):
    cid = lax.axis_index("c")
    sid = lax.axis_index("s")
    k_my = lax.select(cid == 0, K0, K1)
    ng_my = k_my // G
    cstart = lax.select(cid == 0, sid * K0, NS * K0 + sid * K1)

    row0 = sid * RPT
    pltpu.sync_copy(z128_hbm, a_sh.at[pl.ds(row0, RPT)])
    if with_counts:
        pltpu.sync_copy(ones_hbm, ones_v)
        pltpu.sync_copy(z16_hbm, c_sh.at[pl.ds(row0, RPT)])
    plsc.subcore_barrier()

    def pre(g, gb):
        pltpu.async_copy(
            gidx_hbm.at[pl.ds(cstart + g * G, G)], gidx_g.at[gb], isem)
        pltpu.async_copy(
            dst_hbm.at[pl.ds(cstart + g * G, G)], dst_g.at[gb], isem)

    def iwait():
        pltpu.make_async_copy(
            gidx_hbm.at[pl.ds(0, G)], gidx_g.at[0], isem).wait()
        pltpu.make_async_copy(
            dst_hbm.at[pl.ds(0, G)], dst_g.at[0], isem).wait()

    def start(j, b):
        gb = (j // G) % 2
        pltpu.async_copy(y_hbm.at[gidx_g.at[gb, j % G]], rows_v.at[b], gsem)

    def wait(b):
        pltpu.make_async_copy(
            y_hbm.at[gidx_g.at[0, 0]], rows_v.at[b], gsem).wait()

    def ascat(j, b):
        gb = (j // G) % 2
        didx = dst_g.at[gb, j % G]
        pltpu.async_copy(rows_v.at[b], a_sh.at[didx], ssem, add=True)
        if with_counts:
            pltpu.async_copy(ones_v, c_sh.at[didx], ssem, add=True)

    def sdrain():
        pltpu.make_async_copy(rows_v.at[0], a_sh.at[dst_g.at[0, 0]], ssem).wait()
        if with_counts:
            pltpu.make_async_copy(
                ones_v, c_sh.at[dst_g.at[0, 0]], ssem).wait()

    pre(0, 0)
    iwait()
    start(0, 0)

    def mloop(k, carry):
        g = k // (G // 2)
        r = k % (G // 2)
        j0 = 2 * k

        @pl.when(jnp.logical_and(r == 0, g + 1 < ng_my))
        def _():
            pre(g + 1, (g + 1) % 2)

        wait(0)
        start(j0 + 1, 1)
        ascat(j0, 0)

        @pl.when(jnp.logical_and(r == G // 2 - 1, g + 1 < ng_my))
        def _():
            iwait()

        wait(1)
        sdrain()

        @pl.when(j0 + 2 < k_my)
        def _():
            start(j0 + 2, 0)

        ascat(j0 + 1, 1)
        sdrain()
        return carry

    lax.fori_loop(0, k_my // 2, mloop, 0)

    plsc.subcore_barrier()
    pltpu.sync_copy(a_sh.at[pl.ds(row0, RPT)], a_out.at[cid, pl.ds(row0, RPT)])
    if with_counts:
        pltpu.sync_copy(c_sh.at[pl.ds(row0, RPT)],
                        c_out.at[cid, pl.ds(row0, RPT)])


def _sc_body_cnt(gidx_hbm, dst_hbm, y_hbm, z128_hbm, z16_hbm, ones_hbm,
                 a_out, c_out,
                 gidx_g, dst_g, rows_v, ones_v, a_sh, c_sh, gsem, isem, ssem):
    _sc_impl(True, gidx_hbm, dst_hbm, y_hbm, z128_hbm, z16_hbm, ones_hbm,
             a_out, c_out,
             gidx_g, dst_g, rows_v, ones_v, a_sh, c_sh, gsem, isem, ssem)


def _sc_body_nocnt(gidx_hbm, dst_hbm, y_hbm, z128_hbm, z16_hbm, ones_hbm,
                   a_out,
                   gidx_g, dst_g, rows_v, ones_v, a_sh, c_sh, gsem, isem, ssem):
    _sc_impl(False, gidx_hbm, dst_hbm, y_hbm, z128_hbm, z16_hbm, ones_hbm,
             a_out, None,
             gidx_g, dst_g, rows_v, ones_v, a_sh, c_sh, gsem, isem, ssem)


def _sc_aggregate(gidx3, dst3, y, z128, z16, ones16, with_counts):
    mesh = plsc.VectorSubcoreMesh(core_axis_name="c", subcore_axis_name="s")
    if with_counts:
        out_type = (
            jax.ShapeDtypeStruct((NC, NA, D), jnp.float32),
            jax.ShapeDtypeStruct((NC, NA, CW), jnp.float32),
        )
        body = _sc_body_cnt
    else:
        out_type = (jax.ShapeDtypeStruct((NC, NA, D), jnp.float32),)
        body = _sc_body_nocnt
    f = functools.partial(
        pl.kernel,
        out_type=out_type,
        mesh=mesh,
        scratch_types=[
            pltpu.VMEM((2, G, CB), jnp.int32),
            pltpu.VMEM((2, G, CB), jnp.int32),
            pltpu.VMEM((2, CB, D), jnp.float32),
            pltpu.VMEM((CB, CW), jnp.float32),
            pltpu.VMEM_SHARED((NA, D), jnp.float32),
            pltpu.VMEM_SHARED((NA, CW), jnp.float32),
            pltpu.SemaphoreType.DMA,
            pltpu.SemaphoreType.DMA,
            pltpu.SemaphoreType.DMA,
        ],
        compiler_params=pltpu.CompilerParams(use_tc_tiling_on_sc=False),
    )(body)
    return f(gidx3, dst3, y, z128, z16, ones16)


def _post_body(p_ref, pc_ref, x_ref, wsk_ref, bsk_ref, g_ref, be_ref, o_ref):
    s = p_ref[0] + p_ref[1]
    pc = pc_ref[0] + pc_ref[1]
    cnt = jnp.sum(pc, axis=1, keepdims=True) * (1.0 / CW)
    cnt = jnp.maximum(cnt, 1.0)
    h = jnp.maximum(s / cnt, 0.0)
    y = h + jnp.dot(x_ref[...], wsk_ref[...], preferred_element_type=jnp.float32)
    y = y + bsk_ref[...]
    mu = jnp.mean(y, axis=1, keepdims=True)
    d = y - mu
    var = jnp.mean(d * d, axis=1, keepdims=True)
    o_ref[...] = g_ref[...] * (d * lax.rsqrt(var + 1e-5)) + be_ref[...]


def _post(p, pc, x, wskt, bsk, g, be):
    return pl.pallas_call(
        _post_body,
        grid=(N // BN,),
        in_specs=[
            pl.BlockSpec((NC, BN, D), lambda i: (0, i, 0)),
            pl.BlockSpec((NC, BN, CW), lambda i: (0, i, 0)),
            pl.BlockSpec((BN, D), lambda i: (i, 0)),
            pl.BlockSpec((D, D), lambda i: (0, 0)),
            pl.BlockSpec((1, D), lambda i: (0, 0)),
            pl.BlockSpec((1, D), lambda i: (0, 0)),
            pl.BlockSpec((1, D), lambda i: (0, 0)),
        ],
        out_specs=pl.BlockSpec((BN, D), lambda i: (i, 0)),
        out_shape=jax.ShapeDtypeStruct((N, D), jnp.float32),
    )(p, pc, x, wskt, bsk.reshape(1, D), g.reshape(1, D), be.reshape(1, D))


def _layer(x, gidx3, dst3, consts, wt3, b3, wskt, bsk, g, be, pc_prev=None):
    z128, z16, ones16 = consts
    y = _proj(x, wt3, b3).reshape(3 * N, D)
    if pc_prev is None:
        p, pc = _sc_aggregate(gidx3, dst3, y, z128, z16, ones16, True)
    else:
        (p,) = _sc_aggregate(gidx3, dst3, y, z128, z16, ones16, False)
        pc = pc_prev
    return _post(p, pc, x, wskt, bsk, g, be), pc


def kernel(edges, node_emb, W1o, b1o, W1i, b1i, W1s, b1s, g1, be1, Wsk1, bsk1,
           W2o, b2o, W2i, b2i, W2s, b2s, g2, be2, Wsk2, bsk2):
    src = edges[:, 0]
    et = edges[:, 1]
    dst = edges[:, 2]
    pad = EP - E
    srcp = jnp.concatenate([src, jnp.zeros((pad,), jnp.int32)])
    etp = jnp.concatenate([et, jnp.zeros((pad,), jnp.int32)])
    dstp = jnp.concatenate([dst, jnp.full((pad,), N, jnp.int32)])

    gidx3 = _make_gidx(etp, srcp).reshape(TCH, CB)
    dst3 = dstp.reshape(TCH, CB)

    consts = (
        jnp.zeros((RPT, D), jnp.float32),
        jnp.zeros((RPT, CW), jnp.float32),
        jnp.ones((CB, CW), jnp.float32),
    )

    wt1 = jnp.stack([W1o.T, W1i.T, W1s.T])
    b1 = jnp.stack([b1o, b1i, b1s])
    wt2 = jnp.stack([W2o.T, W2i.T, W2s.T])
    b2 = jnp.stack([b2o, b2i, b2s])

    x1, pc = _layer(node_emb, gidx3, dst3, consts, wt1, b1, Wsk1.T, bsk1,
                    g1, be1)
    x2, _ = _layer(x1, gidx3, dst3, consts, wt2, b2, Wsk2.T, bsk2, g2, be2,
                   pc_prev=pc)
    return x2


# speedup vs baseline: 1.8196x; 1.0165x over previous
import functools

import jax
import jax.numpy as jnp
from jax import lax
from jax.experimental import pallas as pl
from jax.experimental.pallas import tpu as pltpu
from jax.experimental.pallas import tpu_sc as plsc

N = 10000
R = 64
D = 128
E = 320000

NC = 2
NS = 16
NW = NC * NS
CB = 112
CH = 90
G = 10
K0 = 130
K1 = 50
TCH = NS * (K0 + K1)
EP = TCH * CB
NA = 10112
RPT = NA // NS
CW = 16
BN = 1000


def _idx_body(et_ref, src_ref, o_ref):
    et = et_ref[...]
    cls = (et >= R).astype(jnp.int32) + (et >= 2 * R).astype(jnp.int32)
    o_ref[...] = cls * N + src_ref[...]


def _make_gidx(etp, srcp):
    rows = EP // 128
    return pl.pallas_call(
        _idx_body,
        out_shape=jax.ShapeDtypeStruct((rows, 128), jnp.int32),
    )(etp.reshape(rows, 128), srcp.reshape(rows, 128))


def _proj_body(x_ref, wt_ref, b_ref, o_ref):
    y = jnp.dot(x_ref[...], wt_ref[0], preferred_element_type=jnp.float32)
    o_ref[...] = (y + b_ref[0])[None]


def _proj(x, wt3, b3):
    return pl.pallas_call(
        _proj_body,
        grid=(3, N // BN),
        in_specs=[
            pl.BlockSpec((BN, D), lambda c, i: (i, 0)),
            pl.BlockSpec((1, D, D), lambda c, i: (c, 0, 0)),
            pl.BlockSpec((1, 1, D), lambda c, i: (c, 0, 0)),
        ],
        out_specs=pl.BlockSpec((1, BN, D), lambda c, i: (c, i, 0)),
        out_shape=jax.ShapeDtypeStruct((3, N, D), jnp.float32),
    )(x, wt3, b3.reshape(3, 1, D))


def _sc_impl(with_counts, gidx_hbm, dst_hbm, y_hbm, z128_hbm, z16_hbm,
             ones_hbm, a_out, c_out,
             gidx_g, dst_g, rows_v, ones_v, a_sh, c_sh,
             gsem, isem, ssem):
    cid = lax.axis_index("c")
    sid = lax.axis_index("s")
    k_my = lax.select(cid == 0, K0, K1)
    ng_my = k_my // G
    cstart = lax.select(cid == 0, sid * K0, NS * K0 + sid * K1)

    row0 = sid * RPT
    pltpu.sync_copy(z128_hbm, a_sh.at[pl.ds(row0, RPT)])
    if with_counts:
        pltpu.sync_copy(ones_hbm, ones_v)
        pltpu.sync_copy(z16_hbm, c_sh.at[pl.ds(row0, RPT)])
    plsc.subcore_barrier()

    def pre(g, gb):
        pltpu.async_copy(
            gidx_hbm.at[pl.ds(cstart + g * G, G)], gidx_g.at[gb], isem)
        pltpu.async_copy(
            dst_hbm.at[pl.ds(cstart + g * G, G)], dst_g.at[gb], isem)

    def iwait():
        pltpu.make_async_copy(
            gidx_hbm.at[pl.ds(0, G)], gidx_g.at[0], isem).wait()
        pltpu.make_async_copy(
            dst_hbm.at[pl.ds(0, G)], dst_g.at[0], isem).wait()

    def start(j, b):
        gb = (j // G) % 2
        pltpu.async_copy(y_hbm.at[gidx_g.at[gb, j % G]], rows_v.at[b], gsem)

    def wait(b):
        pltpu.make_async_copy(
            y_hbm.at[gidx_g.at[0, 0]], rows_v.at[b], gsem).wait()

    def ascat(j, b):
        gb = (j // G) % 2
        didx = dst_g.at[gb, j % G]
        pltpu.async_copy(rows_v.at[b], a_sh.at[didx], ssem, add=True)
        if with_counts:
            pltpu.async_copy(ones_v, c_sh.at[didx], ssem, add=True)

    def sdrain():
        pltpu.make_async_copy(rows_v.at[0], a_sh.at[dst_g.at[0, 0]], ssem).wait()
        if with_counts:
            pltpu.make_async_copy(
                ones_v, c_sh.at[dst_g.at[0, 0]], ssem).wait()

    pre(0, 0)
    iwait()
    start(0, 0)

    def mloop(k, carry):
        g = k // (G // 2)
        r = k % (G // 2)
        j0 = 2 * k

        @pl.when(jnp.logical_and(r == 0, g + 1 < ng_my))
        def _():
            pre(g + 1, (g + 1) % 2)

        wait(0)
        start(j0 + 1, 1)
        ascat(j0, 0)

        @pl.when(jnp.logical_and(r == G // 2 - 1, g + 1 < ng_my))
        def _():
            iwait()

        wait(1)
        sdrain()

        @pl.when(j0 + 2 < k_my)
        def _():
            start(j0 + 2, 0)

        ascat(j0 + 1, 1)
        sdrain()
        return carry

    lax.fori_loop(0, k_my // 2, mloop, 0)

    plsc.subcore_barrier()
    pltpu.sync_copy(a_sh.at[pl.ds(row0, RPT)], a_out.at[cid, pl.ds(row0, RPT)])
    if with_counts:
        pltpu.sync_copy(c_sh.at[pl.ds(row0, RPT)],
                        c_out.at[cid, pl.ds(row0, RPT)])


def _sc_body_cnt(gidx_hbm, dst_hbm, y_hbm, z128_hbm, z16_hbm, ones_hbm,
                 a_out, c_out,
                 gidx_g, dst_g, rows_v, ones_v, a_sh, c_sh, gsem, isem, ssem):
    _sc_impl(True, gidx_hbm, dst_hbm, y_hbm, z128_hbm, z16_hbm, ones_hbm,
             a_out, c_out,
             gidx_g, dst_g, rows_v, ones_v, a_sh, c_sh, gsem, isem, ssem)


def _sc_body_nocnt(gidx_hbm, dst_hbm, y_hbm, z128_hbm, z16_hbm, ones_hbm,
                   a_out,
                   gidx_g, dst_g, rows_v, ones_v, a_sh, c_sh, gsem, isem, ssem):
    _sc_impl(False, gidx_hbm, dst_hbm, y_hbm, z128_hbm, z16_hbm, ones_hbm,
             a_out, None,
             gidx_g, dst_g, rows_v, ones_v, a_sh, c_sh, gsem, isem, ssem)


def _sc_aggregate(gidx3, dst3, y, z128, z16, ones16, with_counts):
    mesh = plsc.VectorSubcoreMesh(core_axis_name="c", subcore_axis_name="s")
    if with_counts:
        out_type = (
            jax.ShapeDtypeStruct((NC, NA, D), jnp.float32),
            jax.ShapeDtypeStruct((NC, NA, CW), jnp.float32),
        )
        body = _sc_body_cnt
    else:
        out_type = (jax.ShapeDtypeStruct((NC, NA, D), jnp.float32),)
        body = _sc_body_nocnt
    f = functools.partial(
        pl.kernel,
        out_type=out_type,
        mesh=mesh,
        scratch_types=[
            pltpu.VMEM((2, G, CB), jnp.int32),
            pltpu.VMEM((2, G, CB), jnp.int32),
            pltpu.VMEM((2, CB, D), jnp.float32),
            pltpu.VMEM((CB, CW), jnp.float32),
            pltpu.VMEM_SHARED((NA, D), jnp.float32),
            pltpu.VMEM_SHARED((NA, CW), jnp.float32),
            pltpu.SemaphoreType.DMA,
            pltpu.SemaphoreType.DMA,
            pltpu.SemaphoreType.DMA,
        ],
        compiler_params=pltpu.CompilerParams(use_tc_tiling_on_sc=False),
    )(body)
    return f(gidx3, dst3, y, z128, z16, ones16)


def _post_body(p_ref, pc_ref, x_ref, wsk_ref, bsk_ref, g_ref, be_ref, o_ref):
    s = p_ref[0] + p_ref[1]
    pc = pc_ref[0] + pc_ref[1]
    cnt = jnp.sum(pc, axis=1, keepdims=True) * (1.0 / CW)
    cnt = jnp.maximum(cnt, 1.0)
    h = jnp.maximum(s / cnt, 0.0)
    y = h + jnp.dot(x_ref[...], wsk_ref[...], preferred_element_type=jnp.float32)
    y = y + bsk_ref[...]
    mu = jnp.mean(y, axis=1, keepdims=True)
    d = y - mu
    var = jnp.mean(d * d, axis=1, keepdims=True)
    o_ref[...] = g_ref[...] * (d * lax.rsqrt(var + 1e-5)) + be_ref[...]


def _post(p, pc, x, wskt, bsk, g, be):
    return pl.pallas_call(
        _post_body,
        grid=(N // BN,),
        in_specs=[
            pl.BlockSpec((NC, BN, D), lambda i: (0, i, 0)),
            pl.BlockSpec((NC, BN, CW), lambda i: (0, i, 0)),
            pl.BlockSpec((BN, D), lambda i: (i, 0)),
            pl.BlockSpec((D, D), lambda i: (0, 0)),
            pl.BlockSpec((1, D), lambda i: (0, 0)),
            pl.BlockSpec((1, D), lambda i: (0, 0)),
            pl.BlockSpec((1, D), lambda i: (0, 0)),
        ],
        out_specs=pl.BlockSpec((BN, D), lambda i: (i, 0)),
        out_shape=jax.ShapeDtypeStruct((N, D), jnp.float32),
    )(p, pc, x, wskt, bsk.reshape(1, D), g.reshape(1, D), be.reshape(1, D))


def _postproj_body(p_ref, pc_ref, x_ref, wsk_ref, bsk_ref, g_ref, be_ref,
                   wt3_ref, b3_ref, x1_ref, y2_ref):
    s = p_ref[0] + p_ref[1]
    pc = pc_ref[0] + pc_ref[1]
    cnt = jnp.sum(pc, axis=1, keepdims=True) * (1.0 / CW)
    cnt = jnp.maximum(cnt, 1.0)
    h = jnp.maximum(s / cnt, 0.0)
    y = h + jnp.dot(x_ref[...], wsk_ref[...], preferred_element_type=jnp.float32)
    y = y + bsk_ref[...]
    mu = jnp.mean(y, axis=1, keepdims=True)
    d = y - mu
    var = jnp.mean(d * d, axis=1, keepdims=True)
    x1 = g_ref[...] * (d * lax.rsqrt(var + 1e-5)) + be_ref[...]
    x1_ref[...] = x1
    for c in range(3):
        y2_ref[c] = (jnp.dot(x1, wt3_ref[c], preferred_element_type=jnp.float32)
                     + b3_ref[c])


def _postproj(p, pc, x, wskt, bsk, g, be, wt3, b3):
    return pl.pallas_call(
        _postproj_body,
        grid=(N // BN,),
        in_specs=[
            pl.BlockSpec((NC, BN, D), lambda i: (0, i, 0)),
            pl.BlockSpec((NC, BN, CW), lambda i: (0, i, 0)),
            pl.BlockSpec((BN, D), lambda i: (i, 0)),
            pl.BlockSpec((D, D), lambda i: (0, 0)),
            pl.BlockSpec((1, D), lambda i: (0, 0)),
            pl.BlockSpec((1, D), lambda i: (0, 0)),
            pl.BlockSpec((1, D), lambda i: (0, 0)),
            pl.BlockSpec((3, D, D), lambda i: (0, 0, 0)),
            pl.BlockSpec((3, 1, D), lambda i: (0, 0, 0)),
        ],
        out_specs=[
            pl.BlockSpec((BN, D), lambda i: (i, 0)),
            pl.BlockSpec((3, BN, D), lambda i: (0, i, 0)),
        ],
        out_shape=[
            jax.ShapeDtypeStruct((N, D), jnp.float32),
            jax.ShapeDtypeStruct((3, N, D), jnp.float32),
        ],
    )(p, pc, x, wskt, bsk.reshape(1, D), g.reshape(1, D), be.reshape(1, D),
      wt3, b3.reshape(3, 1, D))


def _layer(x, gidx3, dst3, consts, wt3, b3, wskt, bsk, g, be, pc_prev=None):
    z128, z16, ones16 = consts
    y = _proj(x, wt3, b3).reshape(3 * N, D)
    if pc_prev is None:
        p, pc = _sc_aggregate(gidx3, dst3, y, z128, z16, ones16, True)
    else:
        (p,) = _sc_aggregate(gidx3, dst3, y, z128, z16, ones16, False)
        pc = pc_prev
    return _post(p, pc, x, wskt, bsk, g, be), pc


def kernel(edges, node_emb, W1o, b1o, W1i, b1i, W1s, b1s, g1, be1, Wsk1, bsk1,
           W2o, b2o, W2i, b2i, W2s, b2s, g2, be2, Wsk2, bsk2):
    src = edges[:, 0]
    et = edges[:, 1]
    dst = edges[:, 2]
    pad = EP - E
    srcp = jnp.concatenate([src, jnp.zeros((pad,), jnp.int32)])
    etp = jnp.concatenate([et, jnp.zeros((pad,), jnp.int32)])
    dstp = jnp.concatenate([dst, jnp.full((pad,), N, jnp.int32)])

    gidx3 = _make_gidx(etp, srcp).reshape(TCH, CB)
    dst3 = dstp.reshape(TCH, CB)

    consts = (
        jnp.zeros((RPT, D), jnp.float32),
        jnp.zeros((RPT, CW), jnp.float32),
        jnp.ones((CB, CW), jnp.float32),
    )

    wt1 = jnp.stack([W1o.T, W1i.T, W1s.T])
    b1 = jnp.stack([b1o, b1i, b1s])
    wt2 = jnp.stack([W2o.T, W2i.T, W2s.T])
    b2 = jnp.stack([b2o, b2i, b2s])

    z128, z16, ones16 = consts
    y1 = _proj(node_emb, wt1, b1).reshape(3 * N, D)
    p1, pc = _sc_aggregate(gidx3, dst3, y1, z128, z16, ones16, True)
    x1, y2 = _postproj(p1, pc, node_emb, Wsk1.T, bsk1, g1, be1, wt2, b2)
    (p2,) = _sc_aggregate(gidx3, dst3, y2.reshape(3 * N, D), z128, z16,
                          ones16, False)
    return _post(p2, pc, x1, Wsk2.T, bsk2, g2, be2)


# speedup vs baseline: 1.8276x; 1.0044x over previous
import functools

import jax
import jax.numpy as jnp
from jax import lax
from jax.experimental import pallas as pl
from jax.experimental.pallas import tpu as pltpu
from jax.experimental.pallas import tpu_sc as plsc

N = 10000
R = 64
D = 128
E = 320000

NC = 2
NS = 16
NW = NC * NS
CB = 112
CH = 90
G = 6
K0 = 132
K1 = 48
TCH = NS * (K0 + K1)
EP = TCH * CB
NA = 10112
RPT = NA // NS
CW = 16
BN = 1000


def _idx_body(et_ref, src_ref, o_ref):
    et = et_ref[...]
    cls = (et >= R).astype(jnp.int32) + (et >= 2 * R).astype(jnp.int32)
    o_ref[...] = cls * N + src_ref[...]


def _make_gidx(etp, srcp):
    rows = EP // 128
    return pl.pallas_call(
        _idx_body,
        out_shape=jax.ShapeDtypeStruct((rows, 128), jnp.int32),
    )(etp.reshape(rows, 128), srcp.reshape(rows, 128))


def _proj_body(x_ref, wt_ref, b_ref, o_ref):
    y = jnp.dot(x_ref[...], wt_ref[0], preferred_element_type=jnp.float32)
    o_ref[...] = (y + b_ref[0])[None]


def _proj(x, wt3, b3):
    return pl.pallas_call(
        _proj_body,
        grid=(3, N // BN),
        in_specs=[
            pl.BlockSpec((BN, D), lambda c, i: (i, 0)),
            pl.BlockSpec((1, D, D), lambda c, i: (c, 0, 0)),
            pl.BlockSpec((1, 1, D), lambda c, i: (c, 0, 0)),
        ],
        out_specs=pl.BlockSpec((1, BN, D), lambda c, i: (c, i, 0)),
        out_shape=jax.ShapeDtypeStruct((3, N, D), jnp.float32),
    )(x, wt3, b3.reshape(3, 1, D))


def _sc_impl(with_counts, gidx_hbm, dst_hbm, y_hbm, z128_hbm, z16_hbm,
             ones_hbm, a_out, c_out,
             gidx_g, dst_g, rows_v, ones_v, a_sh, c_sh,
             gsem, isem, ssem):
    cid = lax.axis_index("c")
    sid = lax.axis_index("s")
    k_my = lax.select(cid == 0, K0, K1)
    ng_my = k_my // G
    cstart = lax.select(cid == 0, sid * K0, NS * K0 + sid * K1)

    row0 = sid * RPT
    pltpu.sync_copy(z128_hbm, a_sh.at[pl.ds(row0, RPT)])
    if with_counts:
        pltpu.sync_copy(ones_hbm, ones_v)
        pltpu.sync_copy(z16_hbm, c_sh.at[pl.ds(row0, RPT)])
    plsc.subcore_barrier()

    def pre(g, gb):
        pltpu.async_copy(
            gidx_hbm.at[pl.ds(cstart + g * G, G)], gidx_g.at[gb], isem)
        pltpu.async_copy(
            dst_hbm.at[pl.ds(cstart + g * G, G)], dst_g.at[gb], isem)

    def iwait():
        pltpu.make_async_copy(
            gidx_hbm.at[pl.ds(0, G)], gidx_g.at[0], isem).wait()
        pltpu.make_async_copy(
            dst_hbm.at[pl.ds(0, G)], dst_g.at[0], isem).wait()

    def start(j, b):
        gb = (j // G) % 2
        pltpu.async_copy(y_hbm.at[gidx_g.at[gb, j % G]], rows_v.at[b], gsem)

    def wait(b):
        pltpu.make_async_copy(
            y_hbm.at[gidx_g.at[0, 0]], rows_v.at[b], gsem).wait()

    def ascat(j, b):
        gb = (j // G) % 2
        didx = dst_g.at[gb, j % G]
        pltpu.async_copy(rows_v.at[b], a_sh.at[didx], ssem, add=True)
        if with_counts:
            pltpu.async_copy(ones_v, c_sh.at[didx], ssem, add=True)

    def sdrain():
        pltpu.make_async_copy(rows_v.at[0], a_sh.at[dst_g.at[0, 0]], ssem).wait()
        if with_counts:
            pltpu.make_async_copy(
                ones_v, c_sh.at[dst_g.at[0, 0]], ssem).wait()

    pre(0, 0)
    iwait()
    start(0, 0)

    def mloop(k, carry):
        g = k // (G // 2)
        r = k % (G // 2)
        j0 = 2 * k

        @pl.when(jnp.logical_and(r == 0, g + 1 < ng_my))
        def _():
            pre(g + 1, (g + 1) % 2)

        wait(0)
        start(j0 + 1, 1)
        ascat(j0, 0)

        @pl.when(jnp.logical_and(r == G // 2 - 1, g + 1 < ng_my))
        def _():
            iwait()

        wait(1)
        sdrain()

        @pl.when(j0 + 2 < k_my)
        def _():
            start(j0 + 2, 0)

        ascat(j0 + 1, 1)
        sdrain()
        return carry

    lax.fori_loop(0, k_my // 2, mloop, 0)

    plsc.subcore_barrier()
    pltpu.sync_copy(a_sh.at[pl.ds(row0, RPT)], a_out.at[cid, pl.ds(row0, RPT)])
    if with_counts:
        pltpu.sync_copy(c_sh.at[pl.ds(row0, RPT)],
                        c_out.at[cid, pl.ds(row0, RPT)])


def _sc_body_cnt(gidx_hbm, dst_hbm, y_hbm, z128_hbm, z16_hbm, ones_hbm,
                 a_out, c_out,
                 gidx_g, dst_g, rows_v, ones_v, a_sh, c_sh, gsem, isem, ssem):
    _sc_impl(True, gidx_hbm, dst_hbm, y_hbm, z128_hbm, z16_hbm, ones_hbm,
             a_out, c_out,
             gidx_g, dst_g, rows_v, ones_v, a_sh, c_sh, gsem, isem, ssem)


def _sc_body_nocnt(gidx_hbm, dst_hbm, y_hbm, z128_hbm, z16_hbm, ones_hbm,
                   a_out,
                   gidx_g, dst_g, rows_v, ones_v, a_sh, c_sh, gsem, isem, ssem):
    _sc_impl(False, gidx_hbm, dst_hbm, y_hbm, z128_hbm, z16_hbm, ones_hbm,
             a_out, None,
             gidx_g, dst_g, rows_v, ones_v, a_sh, c_sh, gsem, isem, ssem)


def _sc_aggregate(gidx3, dst3, y, z128, z16, ones16, with_counts):
    mesh = plsc.VectorSubcoreMesh(core_axis_name="c", subcore_axis_name="s")
    if with_counts:
        out_type = (
            jax.ShapeDtypeStruct((NC, NA, D), jnp.float32),
            jax.ShapeDtypeStruct((NC, NA, CW), jnp.float32),
        )
        body = _sc_body_cnt
    else:
        out_type = (jax.ShapeDtypeStruct((NC, NA, D), jnp.float32),)
        body = _sc_body_nocnt
    f = functools.partial(
        pl.kernel,
        out_type=out_type,
        mesh=mesh,
        scratch_types=[
            pltpu.VMEM((2, G, CB), jnp.int32),
            pltpu.VMEM((2, G, CB), jnp.int32),
            pltpu.VMEM((2, CB, D), jnp.float32),
            pltpu.VMEM((CB, CW), jnp.float32),
            pltpu.VMEM_SHARED((NA, D), jnp.float32),
            pltpu.VMEM_SHARED((NA, CW), jnp.float32),
            pltpu.SemaphoreType.DMA,
            pltpu.SemaphoreType.DMA,
            pltpu.SemaphoreType.DMA,
        ],
        compiler_params=pltpu.CompilerParams(use_tc_tiling_on_sc=False),
    )(body)
    return f(gidx3, dst3, y, z128, z16, ones16)


def _post_body(p_ref, pc_ref, x_ref, wsk_ref, bsk_ref, g_ref, be_ref, o_ref):
    s = p_ref[0] + p_ref[1]
    pc = pc_ref[0] + pc_ref[1]
    cnt = jnp.sum(pc, axis=1, keepdims=True) * (1.0 / CW)
    cnt = jnp.maximum(cnt, 1.0)
    h = jnp.maximum(s / cnt, 0.0)
    y = h + jnp.dot(x_ref[...], wsk_ref[...], preferred_element_type=jnp.float32)
    y = y + bsk_ref[...]
    mu = jnp.mean(y, axis=1, keepdims=True)
    d = y - mu
    var = jnp.mean(d * d, axis=1, keepdims=True)
    o_ref[...] = g_ref[...] * (d * lax.rsqrt(var + 1e-5)) + be_ref[...]


def _post(p, pc, x, wskt, bsk, g, be):
    return pl.pallas_call(
        _post_body,
        grid=(N // BN,),
        in_specs=[
            pl.BlockSpec((NC, BN, D), lambda i: (0, i, 0)),
            pl.BlockSpec((NC, BN, CW), lambda i: (0, i, 0)),
            pl.BlockSpec((BN, D), lambda i: (i, 0)),
            pl.BlockSpec((D, D), lambda i: (0, 0)),
            pl.BlockSpec((1, D), lambda i: (0, 0)),
            pl.BlockSpec((1, D), lambda i: (0, 0)),
            pl.BlockSpec((1, D), lambda i: (0, 0)),
        ],
        out_specs=pl.BlockSpec((BN, D), lambda i: (i, 0)),
        out_shape=jax.ShapeDtypeStruct((N, D), jnp.float32),
    )(p, pc, x, wskt, bsk.reshape(1, D), g.reshape(1, D), be.reshape(1, D))


def _postproj_body(p_ref, pc_ref, x_ref, wsk_ref, bsk_ref, g_ref, be_ref,
                   wt3_ref, b3_ref, x1_ref, y2_ref):
    s = p_ref[0] + p_ref[1]
    pc = pc_ref[0] + pc_ref[1]
    cnt = jnp.sum(pc, axis=1, keepdims=True) * (1.0 / CW)
    cnt = jnp.maximum(cnt, 1.0)
    h = jnp.maximum(s / cnt, 0.0)
    y = h + jnp.dot(x_ref[...], wsk_ref[...], preferred_element_type=jnp.float32)
    y = y + bsk_ref[...]
    mu = jnp.mean(y, axis=1, keepdims=True)
    d = y - mu
    var = jnp.mean(d * d, axis=1, keepdims=True)
    x1 = g_ref[...] * (d * lax.rsqrt(var + 1e-5)) + be_ref[...]
    x1_ref[...] = x1
    for c in range(3):
        y2_ref[c] = (jnp.dot(x1, wt3_ref[c], preferred_element_type=jnp.float32)
                     + b3_ref[c])


def _postproj(p, pc, x, wskt, bsk, g, be, wt3, b3):
    return pl.pallas_call(
        _postproj_body,
        grid=(N // BN,),
        in_specs=[
            pl.BlockSpec((NC, BN, D), lambda i: (0, i, 0)),
            pl.BlockSpec((NC, BN, CW), lambda i: (0, i, 0)),
            pl.BlockSpec((BN, D), lambda i: (i, 0)),
            pl.BlockSpec((D, D), lambda i: (0, 0)),
            pl.BlockSpec((1, D), lambda i: (0, 0)),
            pl.BlockSpec((1, D), lambda i: (0, 0)),
            pl.BlockSpec((1, D), lambda i: (0, 0)),
            pl.BlockSpec((3, D, D), lambda i: (0, 0, 0)),
            pl.BlockSpec((3, 1, D), lambda i: (0, 0, 0)),
        ],
        out_specs=[
            pl.BlockSpec((BN, D), lambda i: (i, 0)),
            pl.BlockSpec((3, BN, D), lambda i: (0, i, 0)),
        ],
        out_shape=[
            jax.ShapeDtypeStruct((N, D), jnp.float32),
            jax.ShapeDtypeStruct((3, N, D), jnp.float32),
        ],
    )(p, pc, x, wskt, bsk.reshape(1, D), g.reshape(1, D), be.reshape(1, D),
      wt3, b3.reshape(3, 1, D))


def _layer(x, gidx3, dst3, consts, wt3, b3, wskt, bsk, g, be, pc_prev=None):
    z128, z16, ones16 = consts
    y = _proj(x, wt3, b3).reshape(3 * N, D)
    if pc_prev is None:
        p, pc = _sc_aggregate(gidx3, dst3, y, z128, z16, ones16, True)
    else:
        (p,) = _sc_aggregate(gidx3, dst3, y, z128, z16, ones16, False)
        pc = pc_prev
    return _post(p, pc, x, wskt, bsk, g, be), pc


def kernel(edges, node_emb, W1o, b1o, W1i, b1i, W1s, b1s, g1, be1, Wsk1, bsk1,
           W2o, b2o, W2i, b2i, W2s, b2s, g2, be2, Wsk2, bsk2):
    src = edges[:, 0]
    et = edges[:, 1]
    dst = edges[:, 2]
    pad = EP - E
    srcp = jnp.concatenate([src, jnp.zeros((pad,), jnp.int32)])
    etp = jnp.concatenate([et, jnp.zeros((pad,), jnp.int32)])
    dstp = jnp.concatenate([dst, jnp.full((pad,), N, jnp.int32)])

    gidx3 = _make_gidx(etp, srcp).reshape(TCH, CB)
    dst3 = dstp.reshape(TCH, CB)

    consts = (
        jnp.zeros((RPT, D), jnp.float32),
        jnp.zeros((RPT, CW), jnp.float32),
        jnp.ones((CB, CW), jnp.float32),
    )

    wt1 = jnp.stack([W1o.T, W1i.T, W1s.T])
    b1 = jnp.stack([b1o, b1i, b1s])
    wt2 = jnp.stack([W2o.T, W2i.T, W2s.T])
    b2 = jnp.stack([b2o, b2i, b2s])

    z128, z16, ones16 = consts
    y1 = _proj(node_emb, wt1, b1).reshape(3 * N, D)
    p1, pc = _sc_aggregate(gidx3, dst3, y1, z128, z16, ones16, True)
    x1, y2 = _postproj(p1, pc, node_emb, Wsk1.T, bsk1, g1, be1, wt2, b2)
    (p2,) = _sc_aggregate(gidx3, dst3, y2.reshape(3 * N, D), z128, z16,
                          ones16, False)
    return _post(p2, pc, x1, Wsk2.T, bsk2, g2, be2)
